# baseline (speedup 1.0000x reference)
import functools

import jax
import jax.numpy as jnp
from jax import lax
from jax.experimental import pallas as pl
from jax.experimental.pallas import tpu as pltpu
from jax.experimental.pallas import tpu_sc as plsc

BATCH = 16384
DIM = 64

_NC = 2
_NS = 16
_NW = _NC * _NS
_BPW = BATCH // _NW
_CHUNK = 128


def _sc_rowdma_body(uids, iids, gu_t, gi_t, out_gu, out_gi,
                    uid_vm, iid_vm, buf_u, buf_i, sem_u, sem_i):
    wid = lax.axis_index("s") * _NC + lax.axis_index("c")
    base = wid * _BPW
    pltpu.sync_copy(uids.at[pl.ds(base, _BPW)], uid_vm)
    pltpu.sync_copy(iids.at[pl.ds(base, _BPW)], iid_vm)

    for c in range(_BPW // _CHUNK):
        def issue(g, _):
            uvec = uid_vm[pl.ds(c * _CHUNK + g * 16, 16)]
            ivec = iid_vm[pl.ds(c * _CHUNK + g * 16, 16)]
            for j in range(16):
                k = g * 16 + j
                pltpu.make_async_copy(
                    gu_t.at[pl.ds(uvec[j], 1)], buf_u.at[pl.ds(k, 1)],
                    sem_u).start()
                pltpu.make_async_copy(
                    gi_t.at[pl.ds(ivec[j], 1)], buf_i.at[pl.ds(k, 1)],
                    sem_i).start()
            return 0

        lax.fori_loop(0, _CHUNK // 16, issue, 0)
        pltpu.make_async_copy(
            gu_t.at[pl.ds(0, _CHUNK)], buf_u, sem_u).wait()
        pltpu.sync_copy(buf_u, out_gu.at[pl.ds(base + c * _CHUNK, _CHUNK)])
        pltpu.make_async_copy(
            gi_t.at[pl.ds(0, _CHUNK)], buf_i, sem_i).wait()
        pltpu.sync_copy(buf_i, out_gi.at[pl.ds(base + c * _CHUNK, _CHUNK)])


@functools.cache
def _make_sc_rowdma():
    return functools.partial(
        pl.kernel,
        out_type=[jax.ShapeDtypeStruct((BATCH, DIM), jnp.float32)] * 2,
        mesh=plsc.VectorSubcoreMesh(core_axis_name="c", subcore_axis_name="s"),
        scratch_types=[
            pltpu.VMEM((_BPW,), jnp.int32),
            pltpu.VMEM((_BPW,), jnp.int32),
            pltpu.VMEM((_CHUNK, DIM), jnp.float32),
            pltpu.VMEM((_CHUNK, DIM), jnp.float32),
            pltpu.SemaphoreType.DMA,
            pltpu.SemaphoreType.DMA,
        ],
    )(_sc_rowdma_body)


def _sc_stream_body(uids, iids, mu_t, mi_t, out_mu, out_mi,
                    uid_vm, iid_vm, buf_u, buf_i, sem_u, sem_i):
    wid = lax.axis_index("s") * _NC + lax.axis_index("c")
    base = wid * _BPW
    pltpu.sync_copy(uids.at[pl.ds(base, _BPW)], uid_vm)
    pltpu.sync_copy(iids.at[pl.ds(base, _BPW)], iid_vm)
    c_u = pltpu.async_copy(mu_t.at[uid_vm], buf_u, sem_u)
    c_i = pltpu.async_copy(mi_t.at[iid_vm], buf_i, sem_i)
    c_u.wait()
    pltpu.sync_copy(buf_u, out_mu.at[pl.ds(base, _BPW)])
    c_i.wait()
    pltpu.sync_copy(buf_i, out_mi.at[pl.ds(base, _BPW)])


@functools.cache
def _make_sc_stream():
    return functools.partial(
        pl.kernel,
        out_type=[jax.ShapeDtypeStruct((BATCH, DIM), jnp.float32)] * 2,
        mesh=plsc.VectorSubcoreMesh(core_axis_name="c", subcore_axis_name="s"),
        compiler_params=pltpu.CompilerParams(use_tc_tiling_on_sc=False),
        scratch_types=[
            pltpu.VMEM((_BPW,), jnp.int32),
            pltpu.VMEM((_BPW,), jnp.int32),
            pltpu.VMEM((_BPW, DIM), jnp.float32),
            pltpu.VMEM((_BPW, DIM), jnp.float32),
            pltpu.SemaphoreType.DMA,
            pltpu.SemaphoreType.DMA,
        ],
    )(_sc_stream_body)


_BLK = 2048


def _tc_mlp_body(gu, gi, mu, mi,
                 w0u, w0i, b0, s0, t0,
                 w1, b1, s1, t1,
                 w2, b2, s2, t2,
                 wg, wx, bo, out):
    x = mu[...] @ w0u[...] + mi[...] @ w0i[...] + b0[...]
    x = jnp.maximum(x, 0.0) * s0[...] + t0[...]
    x = x @ w1[...] + b1[...]
    x = jnp.maximum(x, 0.0) * s1[...] + t1[...]
    x = x @ w2[...] + b2[...]
    x = jnp.maximum(x, 0.0) * s2[...] + t2[...]
    g = gu[...] * gi[...]
    logit = (jnp.sum(g * wg[...], axis=1, keepdims=True)
             + jnp.sum(x * wx[...], axis=1, keepdims=True) + bo[...])
    out[...] = jax.nn.sigmoid(logit)


def _tc_mlp(gu, gi, mu, mi, params):
    n_blk = BATCH // _BLK
    data_spec = pl.BlockSpec((_BLK, DIM), lambda i: (i, 0))

    def full(a):
        return pl.BlockSpec(a.shape, lambda i: (0,) * a.ndim)

    in_specs = [data_spec] * 4 + [full(p) for p in params]
    return pl.pallas_call(
        _tc_mlp_body,
        grid=(n_blk,),
        in_specs=in_specs,
        out_specs=pl.BlockSpec((_BLK, 1), lambda i: (i, 0)),
        out_shape=jax.ShapeDtypeStruct((BATCH, 1), jnp.float32),
    )(gu, gi, mu, mi, *params)


def kernel(inputs, gmf_user_table, gmf_item_table, mlp_user_table, mlp_item_table,
           W0, b0, g0, be0, m0, v0,
           W1, b1, g1, be1, m1, v1,
           W2, b2, g2, be2, m2, v2,
           Wout, bout):
    uids = inputs[:, 0].astype(jnp.int32)
    iids = inputs[:, 1].astype(jnp.int32)

    mu, mi = _make_sc_stream()(uids, iids, mlp_user_table, mlp_item_table)
    gu, gi = _make_sc_rowdma()(uids, iids, gmf_user_table, gmf_item_table)

    def fold(g, be, m, v):
        s = g / jnp.sqrt(v + 1e-3)
        return s, be - m * s

    s0, t0 = fold(g0, be0, m0, v0)
    s1, t1 = fold(g1, be1, m1, v1)
    s2, t2 = fold(g2, be2, m2, v2)

    def row(a):
        return a.reshape(1, -1)

    params = [
        W0[:DIM], W0[DIM:], row(b0), row(s0), row(t0),
        W1, row(b1), row(s1), row(t1),
        W2, row(b2), row(s2), row(t2),
        row(Wout[:DIM, 0]), row(Wout[DIM:, 0]), row(bout),
    ]
    out = _tc_mlp(gu, gi, mu, mi, params)
    return jnp.squeeze(out, axis=1)
